# Initial kernel scaffold; baseline (speedup 1.0000x reference)
import jax
import jax.numpy as jnp
from jax.experimental import pallas as pl


def kernel(x):
    raise NotImplementedError("write your pallas kernel here")


# speedup vs baseline: 1.6102x; 1.6102x over previous
import functools

import jax
import jax.numpy as jnp
from jax import lax
from jax.experimental import pallas as pl
from jax.experimental.pallas import tpu as pltpu
from jax.experimental.pallas import tpu_sc as plsc

ROWS = 128
COLS = 32768
LANES = 16
NUM_WORKERS = 32
ROWS_PER_WORKER = ROWS // NUM_WORKERS
VECS_PER_ROW = COLS // LANES

NEG_INF = float("-inf")


def _insert(a1, a2, a3, v):
    n1 = jnp.maximum(a1, v)
    t = jnp.minimum(a1, v)
    n2 = jnp.maximum(a2, t)
    t2 = jnp.minimum(a2, t)
    n3 = jnp.maximum(a3, t2)
    return n1, n2, n3


@functools.partial(
    pl.kernel,
    mesh=plsc.VectorSubcoreMesh(core_axis_name="c", subcore_axis_name="s"),
    out_type=jax.ShapeDtypeStruct((NUM_WORKERS, LANES), jnp.float32),
    compiler_params=pltpu.CompilerParams(needs_layout_passes=False),
    scratch_types=[
        pltpu.VMEM((COLS,), jnp.float32),
        pltpu.VMEM((COLS,), jnp.float32),
        pltpu.VMEM((LANES,), jnp.float32),
        pltpu.SemaphoreType.DMA,
        pltpu.SemaphoreType.DMA,
    ],
)
def _topk_sc(x_hbm, out_hbm, buf0, buf1, out_v, sem0, sem1):
    wid = lax.axis_index("s") * 2 + lax.axis_index("c")
    base_row = wid * ROWS_PER_WORKER
    bufs = (buf0, buf1)
    sems = (sem0, sem1)
    lanes = lax.iota(jnp.int32, LANES)

    cp0 = pltpu.async_copy(x_hbm.at[base_row], bufs[0], sems[0])
    copies = [cp0]

    res = jnp.zeros((LANES,), jnp.float32)
    for r in range(ROWS_PER_WORKER):
        if r + 1 < ROWS_PER_WORKER:
            copies.append(
                pltpu.async_copy(
                    x_hbm.at[base_row + (r + 1)],
                    bufs[(r + 1) % 2],
                    sems[(r + 1) % 2],
                )
            )
        copies[r].wait()
        buf = bufs[r % 2]

        def body(i, carry, buf=buf):
            a1, a2, a3 = carry
            off = pl.multiple_of(i * LANES, LANES)
            v = buf[pl.ds(off, LANES)]
            return _insert(a1, a2, a3, v)

        init = (
            jnp.full((LANES,), NEG_INF, jnp.float32),
            jnp.full((LANES,), NEG_INF, jnp.float32),
            jnp.full((LANES,), NEG_INF, jnp.float32),
        )
        a1, a2, a3 = lax.fori_loop(0, VECS_PER_ROW, body, init, unroll=8)

        for k in range(3):
            m = jnp.max(a1)
            res = jnp.where(lanes == (3 * r + k), m, res)
            if k < 2:
                f = plsc.all_reduce_ffs(a1 == m)
                sel = lanes == f
                a1 = jnp.where(sel, a2, a1)
                a2 = jnp.where(sel, a3, a2)
                a3 = jnp.where(sel, NEG_INF, a3)

    out_v[...] = res
    pltpu.sync_copy(out_v, out_hbm.at[wid])


def kernel(x):
    staged = _topk_sc(x)
    return staged[:, : 3 * ROWS_PER_WORKER].reshape(ROWS, 3)
